# baseline (speedup 1.0000x reference)
import functools

import jax
import jax.numpy as jnp
from jax import lax
from jax.experimental import pallas as pl
from jax.experimental.pallas import tpu as pltpu
from jax.experimental.pallas import tpu_sc as plsc

N = 10000
D = 128
DH = D // 2
NC, NS, L = 2, 16, 16
B = 128
N_PAD = 10240
RPT = N_PAD // NS


def _sc_agg_body(with_deg, *refs):
    if with_deg:
        (xl_hbm, xr_hbm, cols_hbm, rows_hbm, z2d_hbm, z1d_hbm,
         part_hbm, deg_hbm,
         colsv, rowsv, gbuf, onesv, acc, dacc, isem, zsem, gsem) = refs
    else:
        (xl_hbm, xr_hbm, cols_hbm, rows_hbm, z2d_hbm, part_hbm,
         colsv, rowsv, gbuf, acc, isem, zsem, gsem) = refs

    c = lax.axis_index("c")
    s = lax.axis_index("s")
    row0 = s * RPT
    k_blocks = colsv.shape[0]

    pltpu.async_copy(cols_hbm.at[s], colsv, isem)
    pltpu.async_copy(rows_hbm.at[s], rowsv, isem)

    pltpu.async_copy(z2d_hbm, acc.at[pl.ds(row0, RPT)], zsem)
    if with_deg:
        pltpu.async_copy(z1d_hbm, dacc.at[pl.ds(row0, RPT)], zsem)
        ones16 = jnp.ones((L,), jnp.float32)
        for i in range(B // L):
            onesv[pl.ds(i * L, L)] = ones16

    pltpu.make_async_copy(cols_hbm.at[s], colsv, isem).wait()
    pltpu.make_async_copy(rows_hbm.at[s], rowsv, isem).wait()
    pltpu.make_async_copy(z2d_hbm, acc.at[pl.ds(row0, RPT)], zsem).wait()
    if with_deg:
        pltpu.make_async_copy(z1d_hbm, dacc.at[pl.ds(row0, RPT)],
                              zsem).wait()

    plsc.subcore_barrier()

    def main_loop(xsrc):
        pltpu.async_copy(xsrc.at[colsv.at[0]], gbuf.at[0], gsem)

        def blk(j, carry):
            jb = lax.rem(j, 2)
            pltpu.make_async_copy(xsrc.at[colsv.at[j]], gbuf.at[jb],
                                  gsem).wait()

            @pl.when(j + 1 < k_blocks)
            def _issue():
                pltpu.async_copy(xsrc.at[colsv.at[j + 1]],
                                 gbuf.at[lax.rem(j + 1, 2)], gsem)

            pltpu.sync_copy(gbuf.at[jb], acc.at[rowsv.at[j]], add=True)
            return carry
        lax.fori_loop(0, k_blocks, blk, None)

    @pl.when(c == 0)
    def _left():
        main_loop(xl_hbm)

    @pl.when(c == 1)
    def _right():
        main_loop(xr_hbm)

    if with_deg:
        half = (k_blocks + 1) // 2
        lo = c * half
        hi = jnp.minimum(lo + half, k_blocks)

        def dblk(j, carry):
            pltpu.sync_copy(onesv, dacc.at[rowsv.at[j]], add=True)
            return carry
        lax.fori_loop(lo, hi, dblk, None)

    plsc.subcore_barrier()

    pltpu.sync_copy(acc.at[pl.ds(row0, RPT)],
                    part_hbm.at[c, pl.ds(row0, RPT)])
    if with_deg:
        pltpu.sync_copy(dacc.at[pl.ds(row0, RPT)],
                        deg_hbm.at[c, pl.ds(row0, RPT)])


def _make_sc_agg(k_blocks, with_deg):
    mesh = plsc.VectorSubcoreMesh(core_axis_name="c", subcore_axis_name="s",
                                  num_cores=NC, num_subcores=NS)
    out_type = [jax.ShapeDtypeStruct((NC, N_PAD, DH), jnp.float32)]
    scratch = [
        pltpu.VMEM((k_blocks, B), jnp.int32),
        pltpu.VMEM((k_blocks, B), jnp.int32),
        pltpu.VMEM((2, B, DH), jnp.float32),
    ]
    if with_deg:
        out_type.append(jax.ShapeDtypeStruct((NC, N_PAD), jnp.float32))
        scratch.append(pltpu.VMEM((B,), jnp.float32))
    scratch.append(pltpu.VMEM_SHARED((N_PAD, DH), jnp.float32))
    if with_deg:
        scratch.append(pltpu.VMEM_SHARED((N_PAD,), jnp.float32))
    scratch += [pltpu.SemaphoreType.DMA,
                pltpu.SemaphoreType.DMA,
                pltpu.SemaphoreType.DMA]
    return pl.kernel(
        functools.partial(_sc_agg_body, with_deg),
        out_type=tuple(out_type),
        mesh=mesh,
        scratch_types=scratch,
        compiler_params=pltpu.CompilerParams(use_tc_tiling_on_sc=False),
    )


def _dense_body(x_ref, pl_ref, pr_ref, d0_ref, d1_ref,
                wt_ref, wbl_ref, wbr_ref, b_ref, o_ref):
    inv = 1.0 / jnp.maximum(d0_ref[...] + d1_ref[...], 1.0)
    acc = jnp.dot(x_ref[...], wt_ref[...], preferred_element_type=jnp.float32)
    acc = acc + jnp.dot(pl_ref[...] * inv, wbl_ref[...],
                        preferred_element_type=jnp.float32)
    acc = acc + jnp.dot(pr_ref[...] * inv, wbr_ref[...],
                        preferred_element_type=jnp.float32)
    o_ref[...] = jnp.maximum(acc + b_ref[...], 0.0)


def _dense(x, p_l, p_r, d0, d1, wt, wbl, wbr, b):
    R = 512
    grid = (N_PAD // R,)
    return pl.pallas_call(
        _dense_body,
        grid=grid,
        in_specs=[
            pl.BlockSpec((R, D), lambda i: (i, 0)),
            pl.BlockSpec((R, DH), lambda i: (i, 0)),
            pl.BlockSpec((R, DH), lambda i: (i, 0)),
            pl.BlockSpec((R, 1), lambda i: (i, 0)),
            pl.BlockSpec((R, 1), lambda i: (i, 0)),
            pl.BlockSpec((D, D), lambda i: (0, 0)),
            pl.BlockSpec((DH, D), lambda i: (0, 0)),
            pl.BlockSpec((DH, D), lambda i: (0, 0)),
            pl.BlockSpec((1, D), lambda i: (0, 0)),
        ],
        out_specs=pl.BlockSpec((R, D), lambda i: (i, 0)),
        out_shape=jax.ShapeDtypeStruct((N_PAD, D), jnp.float32),
    )(x, p_l, p_r, d0, d1, wt, wbl, wbr, b)


def kernel(features, rows, cols, W1, b1, W2, b2):
    E = rows.shape[0]
    k_blocks = -(-E // (NS * B))
    e_pad = NS * k_blocks * B

    xpad = jnp.zeros((N_PAD, D), jnp.float32).at[:N, :].set(features)

    pad = e_pad - E
    i = jnp.arange(pad, dtype=jnp.int32)
    cols_p = jnp.concatenate([cols, i % N])
    rows_p = jnp.concatenate([rows, N + i % (N_PAD - N)])
    cols_r = cols_p.reshape(NS, k_blocks, B)
    rows_r = rows_p.reshape(NS, k_blocks, B)

    agg1 = _make_sc_agg(k_blocks, with_deg=True)
    agg2 = _make_sc_agg(k_blocks, with_deg=False)

    z2d = jnp.zeros((RPT, DH), jnp.float32)
    z1d = jnp.zeros((RPT,), jnp.float32)
    part1, degp = agg1(xpad[:, :DH], xpad[:, DH:], cols_r, rows_r, z2d, z1d)
    d0 = degp[0][:, None]
    d1 = degp[1][:, None]
    w1t, w1bl, w1br = W1[:D], W1[D:D + DH], W1[D + DH:]
    w2t, w2bl, w2br = W2[:D], W2[D:D + DH], W2[D + DH:]

    h1 = _dense(xpad, part1[0], part1[1], d0, d1, w1t, w1bl, w1br, b1[None, :])
    (part2,) = agg2(h1[:, :DH], h1[:, DH:], cols_r, rows_r, z2d)
    h2 = _dense(h1, part2[0], part2[1], d0, d1, w2t, w2bl, w2br, b2[None, :])
    return h2[:N]


# speedup vs baseline: 1.2390x; 1.2390x over previous
import functools

import jax
import jax.numpy as jnp
from jax import lax
from jax.experimental import pallas as pl
from jax.experimental.pallas import tpu as pltpu
from jax.experimental.pallas import tpu_sc as plsc

N = 10000
D = 128
DH = D // 2
NC, NS, L = 2, 16, 16
B = 128
RB = 2
N_PAD = 10240
RPT = N_PAD // NS


def _sc_agg_body(with_deg, *refs):
    if with_deg:
        (xl_hbm, xr_hbm, cols_hbm, rows_hbm, z2d_hbm, z1d_hbm,
         part_hbm, deg_hbm,
         colsv, rowsv, gbuf, onesv, acc, dacc, isem, zsem, gsem, ssem) = refs
    else:
        (xl_hbm, xr_hbm, cols_hbm, rows_hbm, z2d_hbm, part_hbm,
         colsv, rowsv, gbuf, acc, isem, zsem, gsem, ssem) = refs

    c = lax.axis_index("c")
    s = lax.axis_index("s")
    row0 = s * RPT
    k_blocks = colsv.shape[0]

    pltpu.async_copy(cols_hbm.at[s], colsv, isem)
    pltpu.async_copy(rows_hbm.at[s], rowsv, isem)

    pltpu.async_copy(z2d_hbm, acc.at[pl.ds(row0, RPT)], zsem)
    if with_deg:
        pltpu.async_copy(z1d_hbm, dacc.at[pl.ds(row0, RPT)], zsem)
        ones16 = jnp.ones((L,), jnp.float32)
        for i in range(RB * B // L):
            onesv[pl.ds(i * L, L)] = ones16

    pltpu.make_async_copy(cols_hbm.at[s], colsv, isem).wait()
    pltpu.make_async_copy(rows_hbm.at[s], rowsv, isem).wait()
    pltpu.make_async_copy(z2d_hbm, acc.at[pl.ds(row0, RPT)], zsem).wait()
    if with_deg:
        pltpu.make_async_copy(z1d_hbm, dacc.at[pl.ds(row0, RPT)],
                              zsem).wait()

    plsc.subcore_barrier()

    n_super = colsv.shape[0]

    def main_loop(xsrc):
        pltpu.async_copy(xsrc.at[colsv.at[0]], gbuf.at[0], gsem.at[0])

        def blk(j, carry):
            b = lax.rem(j, 2)
            pltpu.make_async_copy(xsrc.at[colsv.at[j]],
                                  gbuf.at[b], gsem.at[b]).wait()
            pltpu.async_copy(gbuf.at[b], acc.at[rowsv.at[j]],
                             ssem.at[b], add=True)

            @pl.when(j + 1 < n_super)
            def _issue():
                b2 = lax.rem(j + 1, 2)

                @pl.when(j >= 1)
                def _drain():
                    pltpu.make_async_copy(
                        gbuf.at[b2], acc.at[rowsv.at[0]],
                        ssem.at[b2]).wait()

                pltpu.async_copy(xsrc.at[colsv.at[j + 1]],
                                 gbuf.at[b2], gsem.at[b2])
            return carry
        lax.fori_loop(0, n_super, blk, None)
        for tail in (n_super - 1, n_super - 2):
            if tail >= 0:
                pltpu.make_async_copy(
                    gbuf.at[tail % 2], acc.at[rowsv.at[0]],
                    ssem.at[tail % 2]).wait()

    @pl.when(c == 0)
    def _left():
        main_loop(xl_hbm)

    @pl.when(c == 1)
    def _right():
        main_loop(xr_hbm)

    if with_deg:
        half = (n_super + 1) // 2
        lo = c * half
        hi = jnp.minimum(lo + half, n_super)

        def dblk(j, carry):
            pltpu.sync_copy(onesv, dacc.at[rowsv.at[j]], add=True)
            return carry
        lax.fori_loop(lo, hi, dblk, None)

    plsc.subcore_barrier()

    pltpu.sync_copy(acc.at[pl.ds(row0, RPT)],
                    part_hbm.at[c, pl.ds(row0, RPT)])
    if with_deg:
        pltpu.sync_copy(dacc.at[pl.ds(row0, RPT)],
                        deg_hbm.at[c, pl.ds(row0, RPT)])


def _make_sc_agg(k_blocks, with_deg):
    mesh = plsc.VectorSubcoreMesh(core_axis_name="c", subcore_axis_name="s",
                                  num_cores=NC, num_subcores=NS)
    out_type = [jax.ShapeDtypeStruct((NC, N_PAD, DH), jnp.float32)]
    n_super = k_blocks // RB
    scratch = [
        pltpu.VMEM((n_super, RB * B), jnp.int32),
        pltpu.VMEM((n_super, RB * B), jnp.int32),
        pltpu.VMEM((2, RB * B, DH), jnp.float32),
    ]
    if with_deg:
        out_type.append(jax.ShapeDtypeStruct((NC, N_PAD), jnp.float32))
        scratch.append(pltpu.VMEM((RB * B,), jnp.float32))
    scratch.append(pltpu.VMEM_SHARED((N_PAD, DH), jnp.float32))
    if with_deg:
        scratch.append(pltpu.VMEM_SHARED((N_PAD,), jnp.float32))
    scratch += [pltpu.SemaphoreType.DMA,
                pltpu.SemaphoreType.DMA,
                pltpu.SemaphoreType.DMA((2,)),
                pltpu.SemaphoreType.DMA((2,))]
    return pl.kernel(
        functools.partial(_sc_agg_body, with_deg),
        out_type=tuple(out_type),
        mesh=mesh,
        scratch_types=scratch,
        compiler_params=pltpu.CompilerParams(use_tc_tiling_on_sc=False),
    )


def _dense_body(x_ref, pl_ref, pr_ref, d0_ref, d1_ref,
                wt_ref, wbl_ref, wbr_ref, b_ref, o_ref):
    inv = 1.0 / jnp.maximum(d0_ref[...] + d1_ref[...], 1.0)
    acc = jnp.dot(x_ref[...], wt_ref[...], preferred_element_type=jnp.float32)
    acc = acc + jnp.dot(pl_ref[...] * inv, wbl_ref[...],
                        preferred_element_type=jnp.float32)
    acc = acc + jnp.dot(pr_ref[...] * inv, wbr_ref[...],
                        preferred_element_type=jnp.float32)
    o_ref[...] = jnp.maximum(acc + b_ref[...], 0.0)


def _dense(x, p_l, p_r, d0, d1, wt, wbl, wbr, b):
    R = 512
    grid = (N_PAD // R,)
    return pl.pallas_call(
        _dense_body,
        grid=grid,
        in_specs=[
            pl.BlockSpec((R, D), lambda i: (i, 0)),
            pl.BlockSpec((R, DH), lambda i: (i, 0)),
            pl.BlockSpec((R, DH), lambda i: (i, 0)),
            pl.BlockSpec((R, 1), lambda i: (i, 0)),
            pl.BlockSpec((R, 1), lambda i: (i, 0)),
            pl.BlockSpec((D, D), lambda i: (0, 0)),
            pl.BlockSpec((DH, D), lambda i: (0, 0)),
            pl.BlockSpec((DH, D), lambda i: (0, 0)),
            pl.BlockSpec((1, D), lambda i: (0, 0)),
        ],
        out_specs=pl.BlockSpec((R, D), lambda i: (i, 0)),
        out_shape=jax.ShapeDtypeStruct((N_PAD, D), jnp.float32),
    )(x, p_l, p_r, d0, d1, wt, wbl, wbr, b)


def kernel(features, rows, cols, W1, b1, W2, b2):
    E = rows.shape[0]
    k_blocks = -(-E // (NS * B * RB)) * RB
    e_pad = NS * k_blocks * B

    xpad = jnp.zeros((N_PAD, D), jnp.float32).at[:N, :].set(features)

    pad = e_pad - E
    i = jnp.arange(pad, dtype=jnp.int32)
    cols_p = jnp.concatenate([cols, i % N])
    rows_p = jnp.concatenate([rows, N + i % (N_PAD - N)])
    cols_r = cols_p.reshape(NS, k_blocks // RB, RB * B)
    rows_r = rows_p.reshape(NS, k_blocks // RB, RB * B)

    agg1 = _make_sc_agg(k_blocks, with_deg=True)
    agg2 = _make_sc_agg(k_blocks, with_deg=False)

    z2d = jnp.zeros((RPT, DH), jnp.float32)
    z1d = jnp.zeros((RPT,), jnp.float32)
    part1, degp = agg1(xpad[:, :DH], xpad[:, DH:], cols_r, rows_r, z2d, z1d)
    d0 = degp[0][:, None]
    d1 = degp[1][:, None]
    w1t, w1bl, w1br = W1[:D], W1[D:D + DH], W1[D + DH:]
    w2t, w2bl, w2br = W2[:D], W2[D:D + DH], W2[D + DH:]

    h1 = _dense(xpad, part1[0], part1[1], d0, d1, w1t, w1bl, w1br, b1[None, :])
    (part2,) = agg2(h1[:, :DH], h1[:, DH:], cols_r, rows_r, z2d)
    h2 = _dense(h1, part2[0], part2[1], d0, d1, w2t, w2bl, w2br, b2[None, :])
    return h2[:N]


# speedup vs baseline: 1.2895x; 1.0407x over previous
import functools

import jax
import jax.numpy as jnp
from jax import lax
from jax.experimental import pallas as pl
from jax.experimental.pallas import tpu as pltpu
from jax.experimental.pallas import tpu_sc as plsc

N = 10000
D = 128
NC, NS, L = 2, 16, 16
NW = NC * NS
SB = 128
W = 10
N_PAD = 10240
RPT = N_PAD // NS


def _sc_agg_body(with_deg, n_windows, *refs):
    if with_deg:
        (x_hbm, cols_hbm, rows_hbm, z2d_hbm, z1d_hbm, part_hbm, deg_hbm,
         colsv, rowsv, gbuf, onesv, acc, dacc,
         isem, zsem, gsem, ssem) = refs
    else:
        (x_hbm, cols_hbm, rows_hbm, z2d_hbm, part_hbm,
         colsv, rowsv, gbuf, acc, isem, zsem, gsem, ssem) = refs

    c = lax.axis_index("c")
    s = lax.axis_index("s")
    wid = s * NC + c
    row0 = s * RPT

    def stage(win, slot):
        pltpu.async_copy(cols_hbm.at[wid, win], colsv.at[slot],
                         isem.at[slot])
        pltpu.async_copy(rows_hbm.at[wid, win], rowsv.at[slot],
                         isem.at[slot])

    def stage_wait(win, slot):
        pltpu.make_async_copy(cols_hbm.at[wid, win], colsv.at[slot],
                              isem.at[slot]).wait()
        pltpu.make_async_copy(rows_hbm.at[wid, win], rowsv.at[slot],
                              isem.at[slot]).wait()

    stage(0, 0)

    @pl.when(n_windows > 1)
    def _stage1():
        stage(1, 1)

    pltpu.async_copy(z2d_hbm, acc.at[pl.ds(row0, RPT)], zsem)
    if with_deg:
        pltpu.async_copy(z1d_hbm, dacc.at[pl.ds(row0, RPT)], zsem)
        ones16 = jnp.ones((L,), jnp.float32)
        for i in range(SB // L):
            onesv[pl.ds(i * L, L)] = ones16

    pltpu.make_async_copy(z2d_hbm, acc.at[pl.ds(row0, RPT)], zsem).wait()
    if with_deg:
        pltpu.make_async_copy(z1d_hbm, dacc.at[pl.ds(row0, RPT)],
                              zsem).wait()

    plsc.subcore_barrier()

    stage_wait(0, 0)
    pltpu.async_copy(x_hbm.at[colsv.at[0, 0]], gbuf.at[0], gsem.at[0])

    def window(w, carry):
        p = lax.rem(w, 3)
        p1 = lax.rem(w + 1, 3)
        p2 = lax.rem(w + 2, 3)
        for u in range(W):
            b = u % 2
            b2 = 1 - b
            pltpu.make_async_copy(x_hbm.at[colsv.at[p, u]], gbuf.at[b],
                                  gsem.at[b]).wait()
            pltpu.async_copy(gbuf.at[b], acc.at[rowsv.at[p, u]],
                             ssem.at[b], add=True)
            if with_deg:
                pltpu.sync_copy(onesv, dacc.at[rowsv.at[p, u]], add=True)
            if u == 2:
                @pl.when(w + 2 < n_windows)
                def _prefetch():
                    stage(w + 2, p2)
            if u < W - 1:
                @pl.when(w * W + u >= 1)
                def _drain():
                    pltpu.make_async_copy(gbuf.at[b2],
                                          acc.at[rowsv.at[p, u]],
                                          ssem.at[b2]).wait()
                pltpu.async_copy(x_hbm.at[colsv.at[p, u + 1]], gbuf.at[b2],
                                 gsem.at[b2])
            else:
                @pl.when(w + 1 < n_windows)
                def _next_window():
                    stage_wait(w + 1, p1)
                    pltpu.make_async_copy(gbuf.at[b2],
                                          acc.at[rowsv.at[p, u]],
                                          ssem.at[b2]).wait()
                    pltpu.async_copy(x_hbm.at[colsv.at[p1, 0]], gbuf.at[b2],
                                     gsem.at[b2])
        return carry
    lax.fori_loop(0, n_windows, window, None)

    n_total = n_windows * W
    for tail in (n_total - 1, n_total - 2):
        if tail >= 0:
            pltpu.make_async_copy(gbuf.at[tail % 2], acc.at[rowsv.at[0, 0]],
                                  ssem.at[tail % 2]).wait()

    plsc.subcore_barrier()

    pltpu.sync_copy(acc.at[pl.ds(row0, RPT)],
                    part_hbm.at[c, pl.ds(row0, RPT)])
    if with_deg:
        pltpu.sync_copy(dacc.at[pl.ds(row0, RPT)],
                        deg_hbm.at[c, pl.ds(row0, RPT)])


def _make_sc_agg(n_windows, with_deg):
    mesh = plsc.VectorSubcoreMesh(core_axis_name="c", subcore_axis_name="s",
                                  num_cores=NC, num_subcores=NS)
    out_type = [jax.ShapeDtypeStruct((NC, N_PAD, D), jnp.float32)]
    scratch = [
        pltpu.VMEM((3, W, SB), jnp.int32),
        pltpu.VMEM((3, W, SB), jnp.int32),
        pltpu.VMEM((2, SB, D), jnp.float32),
    ]
    if with_deg:
        out_type.append(jax.ShapeDtypeStruct((NC, N_PAD), jnp.float32))
        scratch.append(pltpu.VMEM((SB,), jnp.float32))
    scratch.append(pltpu.VMEM_SHARED((N_PAD, D), jnp.float32))
    if with_deg:
        scratch.append(pltpu.VMEM_SHARED((N_PAD,), jnp.float32))
    scratch += [pltpu.SemaphoreType.DMA((3,)),
                pltpu.SemaphoreType.DMA,
                pltpu.SemaphoreType.DMA((2,)),
                pltpu.SemaphoreType.DMA((2,))]
    return pl.kernel(
        functools.partial(_sc_agg_body, with_deg, n_windows),
        out_type=tuple(out_type),
        mesh=mesh,
        scratch_types=scratch,
        compiler_params=pltpu.CompilerParams(use_tc_tiling_on_sc=False),
    )


def _dense_body(x_ref, p0_ref, p1_ref, d0_ref, d1_ref,
                wt_ref, wb_ref, b_ref, o_ref):
    inv = 1.0 / jnp.maximum(d0_ref[...] + d1_ref[...], 1.0)
    neigh = (p0_ref[...] + p1_ref[...]) * inv
    acc = jnp.dot(x_ref[...], wt_ref[...], preferred_element_type=jnp.float32)
    acc = acc + jnp.dot(neigh, wb_ref[...],
                        preferred_element_type=jnp.float32)
    o_ref[...] = jnp.maximum(acc + b_ref[...], 0.0)


def _dense(x, p0, p1, d0, d1, wt, wb, b):
    R = 512
    grid = (N_PAD // R,)
    return pl.pallas_call(
        _dense_body,
        grid=grid,
        in_specs=[
            pl.BlockSpec((R, D), lambda i: (i, 0)),
            pl.BlockSpec((R, D), lambda i: (i, 0)),
            pl.BlockSpec((R, D), lambda i: (i, 0)),
            pl.BlockSpec((R, 1), lambda i: (i, 0)),
            pl.BlockSpec((R, 1), lambda i: (i, 0)),
            pl.BlockSpec((D, D), lambda i: (0, 0)),
            pl.BlockSpec((D, D), lambda i: (0, 0)),
            pl.BlockSpec((1, D), lambda i: (0, 0)),
        ],
        out_specs=pl.BlockSpec((R, D), lambda i: (i, 0)),
        out_shape=jax.ShapeDtypeStruct((N_PAD, D), jnp.float32),
    )(x, p0, p1, d0, d1, wt, wb, b)


def kernel(features, rows, cols, W1, b1, W2, b2):
    E = rows.shape[0]
    n_windows = -(-E // (NW * SB * W))
    e_pad = NW * n_windows * W * SB

    xpad = jnp.zeros((N_PAD, D), jnp.float32).at[:N, :].set(features)

    pad = e_pad - E
    i = jnp.arange(pad, dtype=jnp.int32)
    cols_p = jnp.concatenate([cols, i % N])
    rows_p = jnp.concatenate([rows, N + i % (N_PAD - N)])
    cols_r = cols_p.reshape(NW, n_windows, W, SB)
    rows_r = rows_p.reshape(NW, n_windows, W, SB)

    agg1 = _make_sc_agg(n_windows, with_deg=True)
    agg2 = _make_sc_agg(n_windows, with_deg=False)

    z2d = jnp.zeros((RPT, D), jnp.float32)
    z1d = jnp.zeros((RPT,), jnp.float32)
    part1, degp = agg1(xpad, cols_r, rows_r, z2d, z1d)
    d0 = degp[0][:, None]
    d1 = degp[1][:, None]
    w1t, w1b = W1[:D], W1[D:]
    w2t, w2b = W2[:D], W2[D:]

    h1 = _dense(xpad, part1[0], part1[1], d0, d1, w1t, w1b, b1[None, :])
    (part2,) = agg2(h1, cols_r, rows_r, z2d)
    h2 = _dense(h1, part2[0], part2[1], d0, d1, w2t, w2b, b2[None, :])
    return h2[:N]
